# Initial kernel scaffold; baseline (speedup 1.0000x reference)
import jax
import jax.numpy as jnp
from jax.experimental import pallas as pl


def kernel(codes, tables, pos_emb):
    raise NotImplementedError("write your pallas kernel here")


# speedup vs baseline: 1.4863x; 1.4863x over previous
import functools

import jax
import jax.numpy as jnp
from jax import lax
from jax.experimental import pallas as pl
from jax.experimental.pallas import tpu as pltpu
from jax.experimental.pallas import tpu_sc as plsc

_PAD = 1024
_LANES = 16


def _make_sc_lookup(N, K, d, T, n_workers, C):
    rows_per_w = N // n_workers
    n_chunks = rows_per_w // C
    mesh = plsc.VectorSubcoreMesh(core_axis_name="c", subcore_axis_name="s")
    num_cores = mesh.num_cores

    @functools.partial(
        pl.kernel,
        out_type=jax.ShapeDtypeStruct((N, d), jnp.float32),
        mesh=mesh,
        scratch_types=[
            pltpu.VMEM((C * K,), jnp.int32),
            pltpu.VMEM((C * K, d), jnp.float32),
            pltpu.VMEM((C, d), jnp.float32),
            pltpu.SemaphoreType.DMA,
        ],
    )
    def sc_kernel(tab_hbm, idx_hbm, pos_hbm, out_hbm, idx_v, stage_v, out_v, sem):
        wid = lax.axis_index("s") * num_cores + lax.axis_index("c")
        w_base = wid * rows_per_w

        def chunk_body(g, carry):
            base = w_base + g * C
            t0 = lax.rem(base, T)
            pltpu.sync_copy(idx_hbm.at[pl.ds(base * K, C * K)], idx_v)
            pltpu.async_copy(tab_hbm.at[idx_v], stage_v, sem).wait()
            pltpu.sync_copy(pos_hbm.at[pl.ds(t0, C)], out_v)

            def col_body(j, carry2):
                def row_body(r, carry3):
                    acc = out_v[r, pl.ds(j * _LANES, _LANES)]
                    for k in range(K):
                        acc = acc + stage_v[r * K + k, pl.ds(j * _LANES, _LANES)]
                    out_v[r, pl.ds(j * _LANES, _LANES)] = acc
                    return carry3

                return lax.fori_loop(0, C, row_body, carry2, unroll=True)

            lax.fori_loop(0, d // _LANES, col_body, 0)
            pltpu.sync_copy(out_v, out_hbm.at[pl.ds(base, C)])
            return carry

        lax.fori_loop(0, n_chunks, chunk_body, 0)

    return sc_kernel


@jax.jit
def kernel(codes, tables, pos_emb):
    B, K, T = codes.shape
    vocab, d = tables.shape[1], tables.shape[2]
    N = B * T
    tab = tables.at[:, _PAD, :].set(0.0).reshape(K * vocab, d)
    offs = (jnp.arange(K, dtype=jnp.int32) * vocab)[None, None, :]
    idx = (codes.transpose(0, 2, 1) + offs).reshape(N * K)
    sc = _make_sc_lookup(N, K, d, T, n_workers=32, C=8)
    out = sc(tab, idx, pos_emb)
    return out.reshape(B, T, d)


# speedup vs baseline: 1.8153x; 1.2214x over previous
import functools

import jax
import jax.numpy as jnp
from jax import lax
from jax.experimental import pallas as pl
from jax.experimental.pallas import tpu as pltpu
from jax.experimental.pallas import tpu_sc as plsc

_PAD = 1024
_LANES = 16
_NSTAGE = 2
_NOUT = 4


def _make_sc_lookup(N, K, d, T, n_workers, C):
    rows_per_w = N // n_workers
    nc = rows_per_w // C
    assert nc % _NOUT == 0 and nc >= 2 * _NOUT
    mesh = plsc.VectorSubcoreMesh(core_axis_name="c", subcore_axis_name="s")
    num_cores = mesh.num_cores

    @functools.partial(
        pl.kernel,
        out_type=jax.ShapeDtypeStruct((N, d), jnp.float32),
        mesh=mesh,
        scratch_types=[
            pltpu.VMEM((nc, C * K), jnp.int32),
            pltpu.VMEM((_NSTAGE, C * K, d), jnp.float32),
            pltpu.VMEM((_NOUT, C, d), jnp.float32),
            [pltpu.SemaphoreType.DMA] * _NSTAGE,
            [pltpu.SemaphoreType.DMA] * _NOUT,
            [pltpu.SemaphoreType.DMA] * _NOUT,
        ],
    )
    def sc_kernel(tab_hbm, idx_hbm, pos_hbm, out_hbm, idx_v, stage_v, out_v,
                  sem_g, sem_p, sem_w):
        wid = lax.axis_index("s") * num_cores + lax.axis_index("c")
        w_base = wid * rows_per_w
        c_base = wid * nc

        def fire_gather(c, slot):
            pltpu.async_copy(tab_hbm.at[idx_v.at[c]], stage_v.at[slot],
                             sem_g[slot])

        def fire_pos(c, slot):
            t0 = lax.rem(w_base + c * C, T)
            pltpu.async_copy(pos_hbm.at[pl.ds(t0, C)], out_v.at[slot],
                             sem_p[slot])

        def wait_gather(slot):
            pltpu.make_async_copy(tab_hbm.at[idx_v.at[0]], stage_v.at[slot],
                                  sem_g[slot]).wait()

        def wait_pos(slot):
            pltpu.make_async_copy(pos_hbm.at[pl.ds(0, C)], out_v.at[slot],
                                  sem_p[slot]).wait()

        def fire_write(c, slot):
            pltpu.async_copy(out_v.at[slot],
                             out_hbm.at[pl.ds(w_base + c * C, C)], sem_w[slot])

        def wait_write(slot):
            pltpu.make_async_copy(out_v.at[slot], out_hbm.at[pl.ds(0, C)],
                                  sem_w[slot]).wait()

        pltpu.sync_copy(idx_hbm.at[pl.ds(c_base, nc)], idx_v)

        for c0 in range(_NSTAGE):
            fire_gather(c0, c0 % _NSTAGE)
            fire_pos(c0, c0 % _NOUT)

        def outer_body(i, carry):
            g = i * _NOUT
            for b in range(_NOUT):
                c = g + b
                sb = b % _NSTAGE
                wait_gather(sb)
                wait_pos(b)

                def col_body(j, carry2):
                    ds = pl.ds(j * _LANES, _LANES)
                    for r in range(C):
                        acc = out_v[b, r, ds]
                        for k in range(K):
                            acc = acc + stage_v[sb, r * K + k, ds]
                        out_v[b, r, ds] = acc
                    return carry2

                lax.fori_loop(0, d // _LANES, col_body, 0)
                fire_write(c, b)

                nxt = c + _NSTAGE

                @pl.when(nxt < nc)
                def _():
                    fire_gather(nxt, sb)

                    @pl.when(nxt >= _NOUT)
                    def _():
                        wait_write((b + _NSTAGE) % _NOUT)

                    fire_pos(nxt, (b + _NSTAGE) % _NOUT)

            return carry

        lax.fori_loop(0, nc // _NOUT, outer_body, 0)
        for slot in range(_NOUT):
            wait_write(slot)

    return sc_kernel


@jax.jit
def kernel(codes, tables, pos_emb):
    B, K, T = codes.shape
    vocab, d = tables.shape[1], tables.shape[2]
    N = B * T
    C = 4
    tab = tables.at[:, _PAD, :].set(0.0).reshape(K * vocab, d)
    offs = (jnp.arange(K, dtype=jnp.int32) * vocab)[None, None, :]
    idx = (codes.transpose(0, 2, 1) + offs).reshape(N // C, C * K)
    sc = _make_sc_lookup(N, K, d, T, n_workers=32, C=C)
    out = sc(tab, idx, pos_emb)
    return out.reshape(B, T, d)


# speedup vs baseline: 2.7714x; 1.5266x over previous
import functools

import jax
import jax.numpy as jnp
from jax import lax
from jax.experimental import pallas as pl
from jax.experimental.pallas import tpu as pltpu
from jax.experimental.pallas import tpu_sc as plsc

_PAD = 1024
_LANES = 16
_NSTAGE = 2
_NOUT = 4


def _make_sc_lookup(N, K, d, T, n_workers, C):
    rows_per_w = N // n_workers
    nc = rows_per_w // C
    assert nc % _NOUT == 0 and nc >= 2 * _NOUT
    mesh = plsc.VectorSubcoreMesh(core_axis_name="c", subcore_axis_name="s")
    num_cores = mesh.num_cores

    @functools.partial(
        pl.kernel,
        out_type=jax.ShapeDtypeStruct((N, d), jnp.float32),
        mesh=mesh,
        scratch_types=[
            pltpu.VMEM((nc, C * K), jnp.int32),
            pltpu.VMEM((_NSTAGE, C * K, d), jnp.float32),
            pltpu.VMEM((_NOUT, C, d), jnp.float32),
            [pltpu.SemaphoreType.DMA] * _NSTAGE,
            [pltpu.SemaphoreType.DMA] * _NOUT,
            [pltpu.SemaphoreType.DMA] * _NOUT,
        ],
    )
    def sc_kernel(tab_hbm, idx_hbm, pos_hbm, out_hbm, idx_v, stage_v, out_v,
                  sem_g, sem_p, sem_w):
        wid = lax.axis_index("s") * num_cores + lax.axis_index("c")
        w_base = wid * rows_per_w
        c_base = wid * nc

        def fire_gather(c, slot):
            pltpu.async_copy(tab_hbm.at[idx_v.at[c]], stage_v.at[slot],
                             sem_g[slot])

        def fire_pos(c, slot):
            t0 = lax.rem(w_base + c * C, T)
            pltpu.async_copy(pos_hbm.at[pl.ds(t0, C)], out_v.at[slot],
                             sem_p[slot])

        def wait_gather(slot):
            pltpu.make_async_copy(tab_hbm.at[idx_v.at[0]], stage_v.at[slot],
                                  sem_g[slot]).wait()

        def wait_pos(slot):
            pltpu.make_async_copy(pos_hbm.at[pl.ds(0, C)], out_v.at[slot],
                                  sem_p[slot]).wait()

        def fire_write(c, slot):
            pltpu.async_copy(out_v.at[slot],
                             out_hbm.at[pl.ds(w_base + c * C, C)], sem_w[slot])

        def wait_write(slot):
            pltpu.make_async_copy(out_v.at[slot], out_hbm.at[pl.ds(0, C)],
                                  sem_w[slot]).wait()

        pltpu.sync_copy(idx_hbm.at[pl.ds(c_base, nc)], idx_v)

        for c0 in range(_NSTAGE):
            fire_gather(c0, c0 % _NSTAGE)
            fire_pos(c0, c0 % _NOUT)

        def outer_body(i, carry):
            g = i * _NOUT
            for b in range(_NOUT):
                c = g + b
                sb = b % _NSTAGE
                wait_gather(sb)
                wait_pos(b)

                @plsc.parallel_loop(0, d // _LANES, unroll=2)
                def col_body(j):
                    ds = pl.ds(j * _LANES, _LANES)
                    for r in range(C):
                        s = [stage_v[sb, r * K + k, ds] for k in range(K)]
                        t = [s[0] + s[1], s[2] + s[3], s[4] + s[5], s[6] + s[7]]
                        plsc.addupdate(out_v.at[b, r, ds],
                                       (t[0] + t[1]) + (t[2] + t[3]))
                fire_write(c, b)

                nxt = c + _NSTAGE

                @pl.when(nxt < nc)
                def _():
                    fire_gather(nxt, sb)

                    @pl.when(nxt >= _NOUT)
                    def _():
                        wait_write((b + _NSTAGE) % _NOUT)

                    fire_pos(nxt, (b + _NSTAGE) % _NOUT)

            return carry

        lax.fori_loop(0, nc // _NOUT, outer_body, 0)
        for slot in range(_NOUT):
            wait_write(slot)

    return sc_kernel


@jax.jit
def kernel(codes, tables, pos_emb):
    B, K, T = codes.shape
    vocab, d = tables.shape[1], tables.shape[2]
    N = B * T
    C = 4
    tab = tables.at[:, _PAD, :].set(0.0).reshape(K * vocab, d)
    offs = (jnp.arange(K, dtype=jnp.int32) * vocab)[None, None, :]
    idx = (codes.transpose(0, 2, 1) + offs).reshape(N // C, C * K)
    sc = _make_sc_lookup(N, K, d, T, n_workers=32, C=C)
    out = sc(tab, idx, pos_emb)
    return out.reshape(B, T, d)
